# Initial kernel scaffold; baseline (speedup 1.0000x reference)
import jax
import jax.numpy as jnp
from jax.experimental import pallas as pl


def kernel(x, input_emb):
    raise NotImplementedError("write your pallas kernel here")


# speedup vs baseline: 1.0938x; 1.0938x over previous
import functools

import jax
import jax.numpy as jnp
from jax import lax
from jax.experimental import pallas as pl
from jax.experimental.pallas import tpu as pltpu
from jax.experimental.pallas import tpu_sc as plsc

_CHUNK = 1024


def kernel(x, input_emb):
    B, L = x.shape
    V, D = input_emb.shape
    N = B * L

    info = plsc.get_sparse_core_info()
    NC, NS = info.num_cores, info.num_subcores
    NW = NC * NS
    n_per_w = N // NW
    assert n_per_w * NW == N and n_per_w % _CHUNK == 0
    n_chunks = n_per_w // _CHUNK

    idx_flat = x.reshape(N)
    mesh = plsc.VectorSubcoreMesh(core_axis_name="c", subcore_axis_name="s")

    @functools.partial(
        pl.kernel,
        mesh=mesh,
        out_type=jax.ShapeDtypeStruct((N, D), jnp.float32),
        compiler_params=pltpu.CompilerParams(use_tc_tiling_on_sc=False),
        scratch_types=[
            pltpu.VMEM((_CHUNK,), jnp.int32),
            pltpu.VMEM((_CHUNK, D), jnp.float32),
            pltpu.SemaphoreType.DMA,
        ],
    )
    def emb_gather(idx_hbm, table_hbm, out_hbm, idx_v, rows_v, sem):
        wid = lax.axis_index("s") * NC + lax.axis_index("c")
        base = wid * n_per_w

        def body(g, carry):
            off = base + g * _CHUNK
            pltpu.sync_copy(idx_hbm.at[pl.ds(off, _CHUNK)], idx_v)
            pltpu.async_copy(table_hbm.at[idx_v], rows_v, sem).wait()
            pltpu.sync_copy(rows_v, out_hbm.at[pl.ds(off, _CHUNK)])
            return carry

        lax.fori_loop(0, n_chunks, body, 0)

    out = emb_gather(idx_flat, input_emb)
    return out.reshape(B, L, D)


# speedup vs baseline: 1.1120x; 1.0167x over previous
import functools

import jax
import jax.numpy as jnp
from jax import lax
from jax.experimental import pallas as pl
from jax.experimental.pallas import tpu as pltpu
from jax.experimental.pallas import tpu_sc as plsc

_CHUNK = 1600


def kernel(x, input_emb):
    B, L = x.shape
    V, D = input_emb.shape
    N = B * L

    info = plsc.get_sparse_core_info()
    NC, NS = info.num_cores, info.num_subcores
    NW = NC * NS
    n_per_w = N // NW
    assert n_per_w * NW == N and n_per_w % _CHUNK == 0
    n_chunks = n_per_w // _CHUNK

    idx_flat = x.reshape(N)
    mesh = plsc.VectorSubcoreMesh(core_axis_name="c", subcore_axis_name="s")

    @functools.partial(
        pl.kernel,
        mesh=mesh,
        out_type=jax.ShapeDtypeStruct((N, D), jnp.float32),
        compiler_params=pltpu.CompilerParams(use_tc_tiling_on_sc=False),
        scratch_types=[
            pltpu.VMEM((_CHUNK,), jnp.int32),
            pltpu.VMEM((_CHUNK,), jnp.int32),
            pltpu.VMEM((_CHUNK, D), jnp.float32),
            pltpu.VMEM((_CHUNK, D), jnp.float32),
            pltpu.SemaphoreType.DMA,
            pltpu.SemaphoreType.DMA,
            pltpu.SemaphoreType.DMA,
            pltpu.SemaphoreType.DMA,
        ],
    )
    def emb_gather(idx_hbm, table_hbm, out_hbm,
                   idx0, idx1, rows0, rows1, gs0, gs1, os0, os1):
        wid = lax.axis_index("s") * NC + lax.axis_index("c")
        base = wid * n_per_w

        idx_v = (idx0, idx1)
        rows_v = (rows0, rows1)
        gsem = (gs0, gs1)
        osem = (os0, os1)

        gat_desc = [None, None]
        out_desc = [None, None]

        pltpu.sync_copy(idx_hbm.at[pl.ds(base, _CHUNK)], idx_v[0])
        gat_desc[0] = pltpu.async_copy(table_hbm.at[idx_v[0]], rows_v[0], gsem[0])

        for g in range(n_chunks):
            b = g % 2
            nb = 1 - b
            if g + 1 < n_chunks:
                off_n = base + (g + 1) * _CHUNK
                pltpu.sync_copy(idx_hbm.at[pl.ds(off_n, _CHUNK)], idx_v[nb])
                if out_desc[nb] is not None:
                    out_desc[nb].wait()
                gat_desc[nb] = pltpu.async_copy(
                    table_hbm.at[idx_v[nb]], rows_v[nb], gsem[nb])
            gat_desc[b].wait()
            off = base + g * _CHUNK
            out_desc[b] = pltpu.async_copy(
                rows_v[b], out_hbm.at[pl.ds(off, _CHUNK)], osem[b])

        for d in out_desc:
            if d is not None:
                d.wait()

    out = emb_gather(idx_flat, input_emb)
    return out.reshape(B, L, D)
